# baseline (speedup 1.0000x reference)
import functools

import jax
import jax.numpy as jnp
from jax.experimental import pallas as pl
from jax.experimental.pallas import tpu as pltpu

_STRIDES = [8, 16, 32, 64, 128]
_SIZES = [(64, 64), (32, 32), (16, 16), (8, 8), (4, 4)]
_SCORE_THR = 0.05
_NMS_THR = 0.6
_MAX_BOX = 1000
_N = 5456
_NPAD = 5632
_NT = _NPAD // 128
_K = 1024

_NEG = -1e9


def _fcos_body(cls_ref, cen_ref, reg_ref, coord_ref,
               val_ref, clso_ref, box_ref):
    cls_sig = jax.nn.sigmoid(cls_ref[0])
    max_sig = jnp.max(cls_sig, axis=0, keepdims=True)
    ridx = jax.lax.broadcasted_iota(jnp.int32, (80, _NPAD), 0)
    cand = jnp.where(cls_sig == max_sig, ridx, 100000)
    arg = jnp.min(cand, axis=0, keepdims=True)
    cls_f = (arg + 1).astype(jnp.float32)
    cen_sig = jax.nn.sigmoid(cen_ref[0])
    s_row = jnp.sqrt(max_sig * cen_sig)

    cx = coord_ref[0:1, :]
    cy = coord_ref[1:2, :]
    reg = reg_ref[0]
    x1 = cx - reg[0:1, :]
    y1 = cy - reg[1:2, :]
    x2 = cx + reg[2:3, :]
    y2 = cy + reg[3:4, :]

    s2d = s_row.reshape(_NT, 128)
    sT = jnp.transpose(s2d)
    jlt = (jax.lax.broadcasted_iota(jnp.int32, (128, 128), 1)
           < jax.lax.broadcasted_iota(jnp.int32, (128, 128), 0))
    cols = []
    for ti in range(_NT):
        scol = sT[:, ti:ti + 1]
        lo = ti * 128
        hi = lo + 128
        parts = []
        if lo > 0:
            parts.append((s_row[:, :lo] >= scol).astype(jnp.float32))
        sd = s_row[:, lo:hi]
        parts.append(jnp.where((sd > scol) | ((sd == scol) & jlt),
                               1.0, 0.0))
        if hi < _NPAD:
            parts.append((s_row[:, hi:] > scol).astype(jnp.float32))
        ind = jnp.concatenate(parts, axis=1)
        cols.append(jnp.sum(ind, axis=1, keepdims=True))
    racc = jnp.concatenate(cols, axis=1)
    rank_row = jnp.transpose(racc).reshape(1, _NPAD)

    def split3(v):
        a = v.astype(jnp.bfloat16).astype(jnp.float32)
        r1 = v - a
        b = r1.astype(jnp.bfloat16).astype(jnp.float32)
        c = r1 - b
        return [a, b, c]

    parts = []
    for v in (s_row, cls_f, x1, y1, x2, y2):
        parts.extend(split3(v))
    vt = jnp.concatenate(parts, axis=0).astype(jnp.bfloat16)
    riota = jax.lax.broadcasted_iota(jnp.int32, (_K, 1), 0).astype(jnp.float32)
    oh = jnp.where(rank_row == riota, 1.0, 0.0).astype(jnp.bfloat16)
    outc18 = jax.lax.dot_general(oh, vt, (((1,), (1,)), ((), ())),
                                 preferred_element_type=jnp.float32)
    rows18 = jax.lax.dot_general(vt, oh, (((1,), (1,)), ((), ())),
                                 preferred_element_type=jnp.float32)

    def col(i):
        return (outc18[:, 3 * i:3 * i + 1] + outc18[:, 3 * i + 1:3 * i + 2]
                + outc18[:, 3 * i + 2:3 * i + 3])

    def row(i):
        return (rows18[3 * i:3 * i + 1, :] + rows18[3 * i + 1:3 * i + 2, :]
                + rows18[3 * i + 2:3 * i + 3, :])

    vals_col = col(0)
    cls_col = col(1)
    bx = jnp.concatenate([col(2), col(3), col(4), col(5)], axis=1)
    rowi = jax.lax.broadcasted_iota(jnp.int32, (_K, 1), 0)
    in_k = rowi < _MAX_BOX

    maxc = jnp.max(jnp.where(in_k, bx, -3.4e38))
    offc = cls_col * (maxc + 1.0)
    offr = row(1) * (maxc + 1.0)
    x1c = bx[:, 0:1] + offc
    y1c = bx[:, 1:2] + offc
    x2c = bx[:, 2:3] + offc
    y2c = bx[:, 3:4] + offc
    x1r = row(2) + offr
    y1r = row(3) + offr
    x2r = row(4) + offr
    y2r = row(5) + offr
    wx = jnp.maximum(jnp.minimum(x2c, x2r) - jnp.maximum(x1c, x1r), 0.0)
    wy = jnp.maximum(jnp.minimum(y2c, y2r) - jnp.maximum(y1c, y1r), 0.0)
    inter = wx * wy
    areac = jnp.maximum(x2c - x1c, 0.0) * jnp.maximum(y2c - y1c, 0.0)
    arear = jnp.maximum(x2r - x1r, 0.0) * jnp.maximum(y2r - y1r, 0.0)
    union = areac + arear - inter
    iou = inter / (union + 1e-9)
    ci = jax.lax.broadcasted_iota(jnp.int32, (_K, _K), 1)
    rj = jax.lax.broadcasted_iota(jnp.int32, (_K, _K), 0)
    m = jnp.where((iou > _NMS_THR) & (ci < rj), 1.0, 0.0)

    validc = jnp.where((vals_col >= _SCORE_THR) & in_k, 1.0, 0.0)

    def cond(c):
        return ~c[1]

    def body(c):
        keep, _ = c
        sup = jax.lax.dot_general(m, keep, (((1,), (0,)), ((), ())),
                                  preferred_element_type=jnp.float32)
        new = validc * (sup == 0.0).astype(jnp.float32)
        return new, jnp.all(new == keep)

    keep, _ = jax.lax.while_loop(cond, body, (validc, False))

    val_ref[0] = vals_col * keep
    clso_ref[0] = cls_col * keep
    box_ref[0] = bx * keep


def kernel(cls_p3, cls_p4, cls_p5, cls_p6, cls_p7,
           cen_p3, cen_p4, cen_p5, cen_p6, cen_p7,
           reg_p3, reg_p4, reg_p5, reg_p6, reg_p7):
    B = cls_p3.shape[0]
    cls_all = jnp.concatenate(
        [t.reshape(B, 80, -1) for t in (cls_p3, cls_p4, cls_p5, cls_p6, cls_p7)],
        axis=2)
    cen_all = jnp.concatenate(
        [t.reshape(B, 1, -1) for t in (cen_p3, cen_p4, cen_p5, cen_p6, cen_p7)],
        axis=2)
    reg_all = jnp.concatenate(
        [t.reshape(B, 4, -1) for t in (reg_p3, reg_p4, reg_p5, reg_p6, reg_p7)],
        axis=2)
    pad = _NPAD - _N
    cls_all = jnp.pad(cls_all, ((0, 0), (0, 0), (0, pad)), constant_values=_NEG)
    cen_all = jnp.pad(cen_all, ((0, 0), (0, 0), (0, pad)), constant_values=_NEG)
    reg_all = jnp.pad(reg_all, ((0, 0), (0, 0), (0, pad)))

    css = []
    for (h, w), s in zip(_SIZES, _STRIDES):
        xs = (jnp.arange(w, dtype=jnp.float32) + 0.5) * s
        ys = (jnp.arange(h, dtype=jnp.float32) + 0.5) * s
        yy, xx = jnp.meshgrid(ys, xs, indexing='ij')
        css.append(jnp.stack([xx.reshape(-1), yy.reshape(-1)], axis=0))
    coords = jnp.concatenate(css, axis=1)
    coords = jnp.pad(coords, ((0, 0), (0, pad)))

    val, clso, box = pl.pallas_call(
        _fcos_body,
        grid=(B,),
        in_specs=[
            pl.BlockSpec((1, 80, _NPAD), lambda b: (b, 0, 0)),
            pl.BlockSpec((1, 1, _NPAD), lambda b: (b, 0, 0)),
            pl.BlockSpec((1, 4, _NPAD), lambda b: (b, 0, 0)),
            pl.BlockSpec((2, _NPAD), lambda b: (0, 0)),
        ],
        out_specs=[
            pl.BlockSpec((1, _K, 1), lambda b: (b, 0, 0)),
            pl.BlockSpec((1, _K, 1), lambda b: (b, 0, 0)),
            pl.BlockSpec((1, _K, 4), lambda b: (b, 0, 0)),
        ],
        out_shape=[
            jax.ShapeDtypeStruct((B, _K, 1), jnp.float32),
            jax.ShapeDtypeStruct((B, _K, 1), jnp.float32),
            jax.ShapeDtypeStruct((B, _K, 4), jnp.float32),
        ],
        compiler_params=pltpu.CompilerParams(
            dimension_semantics=("parallel",)),
    )(cls_all, cen_all, reg_all, coords)

    return (val[:, :_MAX_BOX, 0],
            clso[:, :_MAX_BOX, 0].astype(jnp.int32),
            box[:, :_MAX_BOX, :])


# speedup vs baseline: 1.1320x; 1.1320x over previous
import functools

import jax
import jax.numpy as jnp
from jax.experimental import pallas as pl
from jax.experimental.pallas import tpu as pltpu

_STRIDES = [8, 16, 32, 64, 128]
_SIZES = [(64, 64), (32, 32), (16, 16), (8, 8), (4, 4)]
_SCORE_THR = 0.05
_NMS_THR = 0.6
_MAX_BOX = 1000
_N = 5456
_NPAD = 5632
_NT = _NPAD // 128
_K = 1024

_NEG = -1e9


def _fcos_body(*refs):
    cls_refs = refs[0:5]
    cen_refs = refs[5:10]
    reg_refs = refs[10:15]
    val_ref, clso_ref, box_ref = refs[15:18]

    s_parts, c_parts, x1_parts, y1_parts, x2_parts, y2_parts = \
        [], [], [], [], [], []
    for l, ((h, w), stride) in enumerate(zip(_SIZES, _STRIDES)):
        hw = h * w
        cls_sig = jax.nn.sigmoid(cls_refs[l][0])
        max_sig = jnp.max(cls_sig, axis=0, keepdims=True)
        ridx = jax.lax.broadcasted_iota(jnp.int32, (80, hw), 0)
        cand = jnp.where(cls_sig == max_sig, ridx, 100000)
        arg = jnp.min(cand, axis=0, keepdims=True)
        cls_f = (arg + 1).astype(jnp.float32)
        cen_sig = jax.nn.sigmoid(cen_refs[l][0])
        s_parts.append(jnp.sqrt(max_sig * cen_sig))
        c_parts.append(cls_f)
        pos = jax.lax.broadcasted_iota(jnp.int32, (1, hw), 1)
        cx = ((pos % w).astype(jnp.float32) + 0.5) * stride
        cy = ((pos // w).astype(jnp.float32) + 0.5) * stride
        reg = reg_refs[l][0]
        x1_parts.append(cx - reg[0:1, :])
        y1_parts.append(cy - reg[1:2, :])
        x2_parts.append(cx + reg[2:3, :])
        y2_parts.append(cy + reg[3:4, :])

    pad = jnp.zeros((1, _NPAD - _N), jnp.float32)
    s_row = jnp.concatenate(s_parts + [pad], axis=1)
    cls_f = jnp.concatenate(c_parts + [pad], axis=1)
    x1 = jnp.concatenate(x1_parts + [pad], axis=1)
    y1 = jnp.concatenate(y1_parts + [pad], axis=1)
    x2 = jnp.concatenate(x2_parts + [pad], axis=1)
    y2 = jnp.concatenate(y2_parts + [pad], axis=1)

    s2d = s_row.reshape(_NT, 128)
    sT = jnp.transpose(s2d)
    jlt = (jax.lax.broadcasted_iota(jnp.int32, (128, 128), 1)
           < jax.lax.broadcasted_iota(jnp.int32, (128, 128), 0))
    cols = []
    for ti in range(_NT):
        scol = sT[:, ti:ti + 1]
        lo = ti * 128
        hi = lo + 128
        parts = []
        if lo > 0:
            parts.append((s_row[:, :lo] >= scol).astype(jnp.float32))
        sd = s_row[:, lo:hi]
        parts.append(jnp.where((sd > scol) | ((sd == scol) & jlt),
                               1.0, 0.0))
        if hi < _NPAD:
            parts.append((s_row[:, hi:] > scol).astype(jnp.float32))
        ind = jnp.concatenate(parts, axis=1)
        cols.append(jnp.sum(ind, axis=1, keepdims=True))
    racc = jnp.concatenate(cols, axis=1)
    rank_row = jnp.transpose(racc).reshape(1, _NPAD)

    def split3(v):
        a = v.astype(jnp.bfloat16).astype(jnp.float32)
        r1 = v - a
        b = r1.astype(jnp.bfloat16).astype(jnp.float32)
        c = r1 - b
        return [a, b, c]

    parts = []
    for v in (s_row, cls_f, x1, y1, x2, y2):
        parts.extend(split3(v))
    vt = jnp.concatenate(parts, axis=0).astype(jnp.bfloat16)
    riota = jax.lax.broadcasted_iota(jnp.int32, (_K, 1), 0).astype(jnp.float32)
    oh = jnp.where(rank_row == riota, 1.0, 0.0).astype(jnp.bfloat16)
    outc18 = jax.lax.dot_general(oh, vt, (((1,), (1,)), ((), ())),
                                 preferred_element_type=jnp.float32)
    rows18 = jax.lax.dot_general(vt, oh, (((1,), (1,)), ((), ())),
                                 preferred_element_type=jnp.float32)

    def col(i):
        return (outc18[:, 3 * i:3 * i + 1] + outc18[:, 3 * i + 1:3 * i + 2]
                + outc18[:, 3 * i + 2:3 * i + 3])

    def row(i):
        return (rows18[3 * i:3 * i + 1, :] + rows18[3 * i + 1:3 * i + 2, :]
                + rows18[3 * i + 2:3 * i + 3, :])

    vals_col = col(0)
    cls_col = col(1)
    bx = jnp.concatenate([col(2), col(3), col(4), col(5)], axis=1)
    rowi = jax.lax.broadcasted_iota(jnp.int32, (_K, 1), 0)
    in_k = rowi < _MAX_BOX

    maxc = jnp.max(jnp.where(in_k, bx, -3.4e38))
    offc = cls_col * (maxc + 1.0)
    offr = row(1) * (maxc + 1.0)
    x1c = bx[:, 0:1] + offc
    y1c = bx[:, 1:2] + offc
    x2c = bx[:, 2:3] + offc
    y2c = bx[:, 3:4] + offc
    x1r = row(2) + offr
    y1r = row(3) + offr
    x2r = row(4) + offr
    y2r = row(5) + offr
    wx = jnp.maximum(jnp.minimum(x2c, x2r) - jnp.maximum(x1c, x1r), 0.0)
    wy = jnp.maximum(jnp.minimum(y2c, y2r) - jnp.maximum(y1c, y1r), 0.0)
    inter = wx * wy
    areac = jnp.maximum(x2c - x1c, 0.0) * jnp.maximum(y2c - y1c, 0.0)
    arear = jnp.maximum(x2r - x1r, 0.0) * jnp.maximum(y2r - y1r, 0.0)
    union = areac + arear - inter
    iou = inter / (union + 1e-9)
    ci = jax.lax.broadcasted_iota(jnp.int32, (_K, _K), 1)
    rj = jax.lax.broadcasted_iota(jnp.int32, (_K, _K), 0)
    m = jnp.where((iou > _NMS_THR) & (ci < rj), 1.0, 0.0)

    validc = jnp.where((vals_col >= _SCORE_THR) & in_k, 1.0, 0.0)

    def cond(c):
        return ~c[1]

    def body(c):
        keep, _ = c
        sup = jax.lax.dot_general(m, keep, (((1,), (0,)), ((), ())),
                                  preferred_element_type=jnp.float32)
        new = validc * (sup == 0.0).astype(jnp.float32)
        return new, jnp.all(new == keep)

    keep, _ = jax.lax.while_loop(cond, body, (validc, False))

    val_ref[0] = vals_col * keep
    clso_ref[0] = cls_col * keep
    box_ref[0] = bx * keep


def kernel(cls_p3, cls_p4, cls_p5, cls_p6, cls_p7,
           cen_p3, cen_p4, cen_p5, cen_p6, cen_p7,
           reg_p3, reg_p4, reg_p5, reg_p6, reg_p7):
    B = cls_p3.shape[0]
    cls_lv = [t.reshape(B, 80, -1)
              for t in (cls_p3, cls_p4, cls_p5, cls_p6, cls_p7)]
    cen_lv = [t.reshape(B, 1, -1)
              for t in (cen_p3, cen_p4, cen_p5, cen_p6, cen_p7)]
    reg_lv = [t.reshape(B, 4, -1)
              for t in (reg_p3, reg_p4, reg_p5, reg_p6, reg_p7)]

    in_specs = (
        [pl.BlockSpec((1, 80, h * w), lambda b: (b, 0, 0))
         for (h, w) in _SIZES]
        + [pl.BlockSpec((1, 1, h * w), lambda b: (b, 0, 0))
           for (h, w) in _SIZES]
        + [pl.BlockSpec((1, 4, h * w), lambda b: (b, 0, 0))
           for (h, w) in _SIZES])

    val, clso, box = pl.pallas_call(
        _fcos_body,
        grid=(B,),
        in_specs=in_specs,
        out_specs=[
            pl.BlockSpec((1, _K, 1), lambda b: (b, 0, 0)),
            pl.BlockSpec((1, _K, 1), lambda b: (b, 0, 0)),
            pl.BlockSpec((1, _K, 4), lambda b: (b, 0, 0)),
        ],
        out_shape=[
            jax.ShapeDtypeStruct((B, _K, 1), jnp.float32),
            jax.ShapeDtypeStruct((B, _K, 1), jnp.float32),
            jax.ShapeDtypeStruct((B, _K, 4), jnp.float32),
        ],
        compiler_params=pltpu.CompilerParams(
            dimension_semantics=("parallel",)),
    )(*cls_lv, *cen_lv, *reg_lv)

    return (val[:, :_MAX_BOX, 0],
            clso[:, :_MAX_BOX, 0].astype(jnp.int32),
            box[:, :_MAX_BOX, :])


# speedup vs baseline: 1.1538x; 1.0193x over previous
import functools

import jax
import jax.numpy as jnp
from jax.experimental import pallas as pl
from jax.experimental.pallas import tpu as pltpu

_STRIDES = [8, 16, 32, 64, 128]
_SIZES = [(64, 64), (32, 32), (16, 16), (8, 8), (4, 4)]
_SCORE_THR = 0.05
_NMS_THR = 0.6
_MAX_BOX = 1000
_N = 5456
_NPAD = 5632
_NT = _NPAD // 128
_K = 1024

_NEG = -1e9


def _fcos_body(*refs):
    cls_refs = refs[0:5]
    cen_refs = refs[5:10]
    reg_refs = refs[10:15]
    val_ref, clso_ref, box_ref = refs[15:18]

    s_parts, c_parts, x1_parts, y1_parts, x2_parts, y2_parts = \
        [], [], [], [], [], []
    for l, ((h, w), stride) in enumerate(zip(_SIZES, _STRIDES)):
        hw = h * w
        cls_sig = jax.nn.sigmoid(cls_refs[l][0])
        max_sig = jnp.max(cls_sig, axis=0, keepdims=True)
        ridx = jax.lax.broadcasted_iota(jnp.int32, (80, hw), 0)
        cand = jnp.where(cls_sig == max_sig, ridx, 100000)
        arg = jnp.min(cand, axis=0, keepdims=True)
        cls_f = (arg + 1).astype(jnp.float32)
        cen_sig = jax.nn.sigmoid(cen_refs[l][0])
        s_parts.append(jnp.sqrt(max_sig * cen_sig))
        c_parts.append(cls_f)
        pos = jax.lax.broadcasted_iota(jnp.int32, (1, hw), 1)
        cx = ((pos % w).astype(jnp.float32) + 0.5) * stride
        cy = ((pos // w).astype(jnp.float32) + 0.5) * stride
        reg = reg_refs[l][0]
        x1_parts.append(cx - reg[0:1, :])
        y1_parts.append(cy - reg[1:2, :])
        x2_parts.append(cx + reg[2:3, :])
        y2_parts.append(cy + reg[3:4, :])

    pad = jnp.zeros((1, _NPAD - _N), jnp.float32)
    s_row = jnp.concatenate(s_parts + [pad], axis=1)
    cls_f = jnp.concatenate(c_parts + [pad], axis=1)
    x1 = jnp.concatenate(x1_parts + [pad], axis=1)
    y1 = jnp.concatenate(y1_parts + [pad], axis=1)
    x2 = jnp.concatenate(x2_parts + [pad], axis=1)
    y2 = jnp.concatenate(y2_parts + [pad], axis=1)

    _C = 512
    n_chunks = _NPAD // _C
    rgt = jax.lax.broadcasted_iota(jnp.int32, (_C, _C), 1)
    rlt = jax.lax.broadcasted_iota(jnp.int32, (_C, _C), 0)
    dmask_gt = rgt > rlt
    dmask_eq = rgt == rlt
    ones_col = jnp.ones((_C, 1), jnp.float32)
    ones_row = jnp.ones((1, _C), jnp.float32)
    acc_row = jnp.zeros((1, _NPAD), jnp.float32)
    rank_parts = []
    for c in range(n_chunks):
        lo = c * _C
        hi = lo + _C
        scol = jnp.transpose(s_row[:, lo:hi])
        sd = s_row[:, lo:hi]
        gtf = (sd > scol).astype(jnp.float32)
        gef = (sd >= scol).astype(jnp.float32)
        ind = jnp.where(dmask_gt, gtf, gef)
        ind = jnp.where(dmask_eq, 0.0, ind)
        d1 = jax.lax.dot_general(ind, ones_col, (((1,), (0,)), ((), ())),
                                 preferred_element_type=jnp.float32)
        if hi < _NPAD:
            wa = _NPAD - hi
            gt_a = (s_row[:, hi:] > scol).astype(jnp.float32)
            d2 = jax.lax.dot_general(
                gt_a, jnp.ones((wa, 1), jnp.float32), (((1,), (0,)), ((), ())),
                preferred_element_type=jnp.float32)
            cs = jax.lax.dot_general(ones_row, gt_a, (((1,), (0,)), ((), ())),
                                     preferred_element_type=jnp.float32)
            acc_row = acc_row + jnp.concatenate(
                [jnp.zeros((1, hi), jnp.float32), float(_C) - cs], axis=1)
            d1 = d1 + d2
        rank_parts.append(jnp.transpose(d1))
    rank_row = jnp.concatenate(rank_parts, axis=1) + acc_row

    def split3(v):
        a = v.astype(jnp.bfloat16).astype(jnp.float32)
        r1 = v - a
        b = r1.astype(jnp.bfloat16).astype(jnp.float32)
        c = r1 - b
        return [a, b, c]

    parts = []
    for v in (s_row, cls_f, x1, y1, x2, y2):
        parts.extend(split3(v))
    vt = jnp.concatenate(parts, axis=0).astype(jnp.bfloat16)
    riota = jax.lax.broadcasted_iota(jnp.int32, (_K, 1), 0).astype(jnp.float32)
    oh = jnp.where(rank_row == riota, 1.0, 0.0).astype(jnp.bfloat16)
    outc18 = jax.lax.dot_general(oh, vt, (((1,), (1,)), ((), ())),
                                 preferred_element_type=jnp.float32)
    rows18 = jax.lax.dot_general(vt, oh, (((1,), (1,)), ((), ())),
                                 preferred_element_type=jnp.float32)

    def col(i):
        return (outc18[:, 3 * i:3 * i + 1] + outc18[:, 3 * i + 1:3 * i + 2]
                + outc18[:, 3 * i + 2:3 * i + 3])

    def row(i):
        return (rows18[3 * i:3 * i + 1, :] + rows18[3 * i + 1:3 * i + 2, :]
                + rows18[3 * i + 2:3 * i + 3, :])

    vals_col = col(0)
    cls_col = col(1)
    bx = jnp.concatenate([col(2), col(3), col(4), col(5)], axis=1)
    rowi = jax.lax.broadcasted_iota(jnp.int32, (_K, 1), 0)
    in_k = rowi < _MAX_BOX

    maxc = jnp.max(jnp.where(in_k, bx, -3.4e38))
    offc = cls_col * (maxc + 1.0)
    offr = row(1) * (maxc + 1.0)
    x1c = bx[:, 0:1] + offc
    y1c = bx[:, 1:2] + offc
    x2c = bx[:, 2:3] + offc
    y2c = bx[:, 3:4] + offc
    x1r = row(2) + offr
    y1r = row(3) + offr
    x2r = row(4) + offr
    y2r = row(5) + offr
    wx = jnp.maximum(jnp.minimum(x2c, x2r) - jnp.maximum(x1c, x1r), 0.0)
    wy = jnp.maximum(jnp.minimum(y2c, y2r) - jnp.maximum(y1c, y1r), 0.0)
    inter = wx * wy
    areac = jnp.maximum(x2c - x1c, 0.0) * jnp.maximum(y2c - y1c, 0.0)
    arear = jnp.maximum(x2r - x1r, 0.0) * jnp.maximum(y2r - y1r, 0.0)
    union = areac + arear - inter
    iou = inter / (union + 1e-9)
    ci = jax.lax.broadcasted_iota(jnp.int32, (_K, _K), 1)
    rj = jax.lax.broadcasted_iota(jnp.int32, (_K, _K), 0)
    m = jnp.where((iou > _NMS_THR) & (ci < rj), 1.0, 0.0)

    validc = jnp.where((vals_col >= _SCORE_THR) & in_k, 1.0, 0.0)

    def cond(c):
        return ~c[1]

    def body(c):
        keep, _ = c
        sup = jax.lax.dot_general(m, keep, (((1,), (0,)), ((), ())),
                                  preferred_element_type=jnp.float32)
        new = validc * (sup == 0.0).astype(jnp.float32)
        return new, jnp.all(new == keep)

    keep, _ = jax.lax.while_loop(cond, body, (validc, False))

    val_ref[0] = vals_col * keep
    clso_ref[0] = cls_col * keep
    box_ref[0] = bx * keep


def kernel(cls_p3, cls_p4, cls_p5, cls_p6, cls_p7,
           cen_p3, cen_p4, cen_p5, cen_p6, cen_p7,
           reg_p3, reg_p4, reg_p5, reg_p6, reg_p7):
    B = cls_p3.shape[0]
    cls_lv = [t.reshape(B, 80, -1)
              for t in (cls_p3, cls_p4, cls_p5, cls_p6, cls_p7)]
    cen_lv = [t.reshape(B, 1, -1)
              for t in (cen_p3, cen_p4, cen_p5, cen_p6, cen_p7)]
    reg_lv = [t.reshape(B, 4, -1)
              for t in (reg_p3, reg_p4, reg_p5, reg_p6, reg_p7)]

    in_specs = (
        [pl.BlockSpec((1, 80, h * w), lambda b: (b, 0, 0))
         for (h, w) in _SIZES]
        + [pl.BlockSpec((1, 1, h * w), lambda b: (b, 0, 0))
           for (h, w) in _SIZES]
        + [pl.BlockSpec((1, 4, h * w), lambda b: (b, 0, 0))
           for (h, w) in _SIZES])

    val, clso, box = pl.pallas_call(
        _fcos_body,
        grid=(B,),
        in_specs=in_specs,
        out_specs=[
            pl.BlockSpec((1, _K, 1), lambda b: (b, 0, 0)),
            pl.BlockSpec((1, _K, 1), lambda b: (b, 0, 0)),
            pl.BlockSpec((1, _K, 4), lambda b: (b, 0, 0)),
        ],
        out_shape=[
            jax.ShapeDtypeStruct((B, _K, 1), jnp.float32),
            jax.ShapeDtypeStruct((B, _K, 1), jnp.float32),
            jax.ShapeDtypeStruct((B, _K, 4), jnp.float32),
        ],
        compiler_params=pltpu.CompilerParams(
            dimension_semantics=("parallel",)),
    )(*cls_lv, *cen_lv, *reg_lv)

    return (val[:, :_MAX_BOX, 0],
            clso[:, :_MAX_BOX, 0].astype(jnp.int32),
            box[:, :_MAX_BOX, :])
